# baseline (speedup 1.0000x reference)
import functools
import math

import jax
import jax.numpy as jnp
import numpy as np
from jax.experimental import pallas as pl
from jax.experimental.pallas import tpu as pltpu
from jax.experimental.pallas import tpu_sc as plsc

_N = 2048
_D = 256
_SIZE = 128
_ROWS = 128

_NC = 2
_NS = 16
_NW = _NC * _NS
_RPW = _N // _NW
_LANES = 16

_INV_SQRT_2PI = 1.0 / math.sqrt(2.0 * math.pi)


def _result_block_tc(x_ref, sel_ref, mask_ref, evg_ref, mu_ref, sig_ref,
                     out_ref, probs_ref):
    xb = x_ref[...]
    sel = sel_ref[...]
    ev_vals = jnp.dot(xb, sel, preferred_element_type=jnp.float32)

    evg = evg_ref[...] > 0.0
    mu = mu_ref[...]
    sigma = sig_ref[...]
    inv = _INV_SQRT_2PI / sigma
    z = (ev_vals - mu) / sigma
    pdf = jnp.exp(-0.5 * z * z) * inv
    res = jnp.where(evg, ev_vals, mu)
    probs_ref[...] = jnp.where(evg, pdf, inv)

    out_ref[...] = res[:, :, None] * mask_ref[...][None, :, :]


def _probs_sc_body(x_hbm, evg_hbm, mu_hbm, sig_hbm, out_hbm,
                   x_v, evg_v, mu_v, sig_v, out_v):
    cid = jax.lax.axis_index("c")
    sid = jax.lax.axis_index("s")
    wid = sid * _NC + cid
    base = wid * _RPW

    pltpu.sync_copy(x_hbm.at[pl.ds(base, _RPW)], x_v)
    pltpu.sync_copy(evg_hbm, evg_v)
    pltpu.sync_copy(mu_hbm, mu_v)
    pltpu.sync_copy(sig_hbm, sig_v)

    ngroups = _SIZE // _LANES
    mu_g = [mu_v[pl.ds(g * _LANES, _LANES)] for g in range(ngroups)]
    rsig_g = [1.0 / sig_v[pl.ds(g * _LANES, _LANES)] for g in range(ngroups)]
    inv_g = [_INV_SQRT_2PI * rs for rs in rsig_g]
    evb_g = [evg_v[pl.ds(g * _LANES, _LANES)] > 0.0 for g in range(ngroups)]

    lane = jax.lax.iota(jnp.int32, _LANES)
    lane_is_lo = lane < (_LANES // 2)
    idx_lo = jnp.where(lane_is_lo, 2 * lane, 0)
    idx_hi = jnp.where(lane_is_lo, 0, 2 * lane - _LANES)
    dnums = jax.lax.GatherDimensionNumbers(
        offset_dims=(), collapsed_slice_dims=(0,), start_index_map=(0,))

    def vgather(v, idx):
        return jax.lax.gather(
            v, idx[:, None], dnums, slice_sizes=(1,),
            mode=jax.lax.GatherScatterMode.PROMISE_IN_BOUNDS)

    def row_body(r, carry):
        for g in range(ngroups):
            v0 = x_v[r, pl.ds(g * 2 * _LANES, _LANES)]
            v1 = x_v[r, pl.ds(g * 2 * _LANES + _LANES, _LANES)]
            v = jnp.where(lane_is_lo, vgather(v0, idx_lo), vgather(v1, idx_hi))
            z = (v - mu_g[g]) * rsig_g[g]
            pdf = jnp.exp(-0.5 * z * z) * inv_g[g]
            out_v[r, pl.ds(g * _LANES, _LANES)] = jnp.where(evb_g[g], pdf,
                                                            inv_g[g])
        return carry

    jax.lax.fori_loop(0, _RPW, row_body, 0)

    for j in range(_KDUP):
        pltpu.sync_copy(out_v, out_hbm.at[j].at[pl.ds(base, _RPW)])


_KDUP = 32

_probs_sc = functools.partial(
    pl.kernel,
    out_type=jax.ShapeDtypeStruct((_KDUP, _N, _SIZE), jnp.float32),
    mesh=plsc.VectorSubcoreMesh(core_axis_name="c", subcore_axis_name="s"),
    scratch_types=[
        pltpu.VMEM((_RPW, _D), jnp.float32),
        pltpu.VMEM((_SIZE,), jnp.float32),
        pltpu.VMEM((_SIZE,), jnp.float32),
        pltpu.VMEM((_SIZE,), jnp.float32),
        pltpu.VMEM((_RPW, _SIZE), jnp.float32),
    ],
)(_probs_sc_body)


def kernel(x, evidence, mu, sigma):
    n, d = x.shape
    size = mu.shape[0]

    sel = np.zeros((d, size), dtype=np.float32)
    sel[np.arange(size) * 2, np.arange(size)] = 1.0
    sel = jnp.asarray(sel)
    mask = np.zeros((size, d), dtype=np.float32)
    mask[np.arange(size), np.arange(size) * 2] = 1.0
    mask = jnp.asarray(mask)

    evg_flat = evidence[::2].astype(jnp.float32)
    evg = evg_flat.reshape(1, size)
    mu2 = mu.reshape(1, size)

    grid = (n // _ROWS,)
    out = pl.pallas_call(
        _result_block_tc,
        grid=grid,
        in_specs=[
            pl.BlockSpec((_ROWS, d), lambda i: (i, 0)),
            pl.BlockSpec((d, size), lambda i: (0, 0)),
            pl.BlockSpec((size, d), lambda i: (0, 0)),
            pl.BlockSpec((1, size), lambda i: (0, 0)),
            pl.BlockSpec((1, size), lambda i: (0, 0)),
            pl.BlockSpec((1, size), lambda i: (0, 0)),
        ],
        out_specs=[
            pl.BlockSpec((_ROWS, size, d), lambda i: (i, 0, 0)),
            pl.BlockSpec((_ROWS, size), lambda i: (i, 0)),
        ],
        out_shape=[
            jax.ShapeDtypeStruct((n, size, d), x.dtype),
            jax.ShapeDtypeStruct((n, size), x.dtype),
        ],
    )(x, sel, mask, evg, mu2, sigma.reshape(1, size))
    out, _tc_probs = out
    probs = _probs_sc(x, evg_flat, mu, sigma)[0]
    return out, probs


# speedup vs baseline: 1.2882x; 1.2882x over previous
import math

import jax
import jax.numpy as jnp
import numpy as np
from jax.experimental import pallas as pl

_N = 2048
_D = 256
_SIZE = 128
_ROWS = 128

_INV_SQRT_2PI = 1.0 / math.sqrt(2.0 * math.pi)


def _leaf_block(x_ref, sel_ref, mask_ref, ev_ref, mu_ref, sig_ref,
                out_ref, probs_ref):
    xb = x_ref[...]
    sel = sel_ref[...]
    ev_vals = jnp.dot(xb, sel, preferred_element_type=jnp.float32)

    evg = jnp.dot(ev_ref[...], sel,
                  preferred_element_type=jnp.float32) > 0.0
    mu = mu_ref[...]
    sigma = sig_ref[...]
    inv = _INV_SQRT_2PI / sigma
    z = (ev_vals - mu) / sigma
    pdf = jnp.exp(-0.5 * z * z) * inv
    res = jnp.where(evg, ev_vals, mu)
    probs_ref[...] = jnp.where(evg, pdf, inv)

    out_ref[...] = res[:, :, None] * mask_ref[...][None, :, :]


def kernel(x, evidence, mu, sigma):
    n, d = x.shape
    size = mu.shape[0]

    sel = np.zeros((d, size), dtype=np.float32)
    sel[np.arange(size) * 2, np.arange(size)] = 1.0
    sel = jnp.asarray(sel)
    mask = np.zeros((size, d), dtype=np.float32)
    mask[np.arange(size), np.arange(size) * 2] = 1.0
    mask = jnp.asarray(mask)

    ev2 = evidence.astype(jnp.float32).reshape(1, d)
    mu2 = mu.reshape(1, size)
    sig2 = sigma.reshape(1, size)

    grid = (n // _ROWS,)
    out, probs = pl.pallas_call(
        _leaf_block,
        grid=grid,
        in_specs=[
            pl.BlockSpec((_ROWS, d), lambda i: (i, 0)),
            pl.BlockSpec((d, size), lambda i: (0, 0)),
            pl.BlockSpec((size, d), lambda i: (0, 0)),
            pl.BlockSpec((1, d), lambda i: (0, 0)),
            pl.BlockSpec((1, size), lambda i: (0, 0)),
            pl.BlockSpec((1, size), lambda i: (0, 0)),
        ],
        out_specs=[
            pl.BlockSpec((_ROWS, size, d), lambda i: (i, 0, 0)),
            pl.BlockSpec((_ROWS, size), lambda i: (i, 0)),
        ],
        out_shape=[
            jax.ShapeDtypeStruct((n, size, d), x.dtype),
            jax.ShapeDtypeStruct((n, size), x.dtype),
        ],
    )(x, sel, mask, ev2, mu2, sig2)
    return out, probs


# speedup vs baseline: 1.3197x; 1.0245x over previous
import math

import jax
import jax.numpy as jnp
import numpy as np
from jax.experimental import pallas as pl

_N = 2048
_D = 256
_SIZE = 128
_ROWS = 128

_INV_SQRT_2PI = 1.0 / math.sqrt(2.0 * math.pi)


def _leaf_block(x_ref, sel_ref, mask_ref, evg_ref, mu_ref, sig_ref,
                out_ref, probs_ref):
    xb = x_ref[...]
    sel = sel_ref[...]
    ev_vals = jnp.dot(xb, sel, preferred_element_type=jnp.float32)

    evg = evg_ref[...] > 0.0
    mu = mu_ref[...]
    sigma = sig_ref[...]
    inv = _INV_SQRT_2PI / sigma
    z = (ev_vals - mu) / sigma
    pdf = jnp.exp(-0.5 * z * z) * inv
    res = jnp.where(evg, ev_vals, mu)
    probs_ref[...] = jnp.where(evg, pdf, inv)

    out_ref[...] = res[:, :, None] * mask_ref[...][None, :, :]


def kernel(x, evidence, mu, sigma):
    n, d = x.shape
    size = mu.shape[0]

    sel = np.zeros((d, size), dtype=np.float32)
    sel[np.arange(size) * 2, np.arange(size)] = 1.0
    sel = jnp.asarray(sel)
    mask = np.zeros((size, d), dtype=np.float32)
    mask[np.arange(size), np.arange(size) * 2] = 1.0
    mask = jnp.asarray(mask)

    evg2 = evidence[::2].astype(jnp.float32).reshape(1, size)
    mu2 = mu.reshape(1, size)
    sig2 = sigma.reshape(1, size)

    grid = (n // _ROWS,)
    out, probs = pl.pallas_call(
        _leaf_block,
        grid=grid,
        in_specs=[
            pl.BlockSpec((_ROWS, d), lambda i: (i, 0)),
            pl.BlockSpec((d, size), lambda i: (0, 0)),
            pl.BlockSpec((size, d), lambda i: (0, 0)),
            pl.BlockSpec((1, size), lambda i: (0, 0)),
            pl.BlockSpec((1, size), lambda i: (0, 0)),
            pl.BlockSpec((1, size), lambda i: (0, 0)),
        ],
        out_specs=[
            pl.BlockSpec((_ROWS, size, d), lambda i: (i, 0, 0)),
            pl.BlockSpec((_ROWS, size), lambda i: (i, 0)),
        ],
        out_shape=[
            jax.ShapeDtypeStruct((n, size, d), x.dtype),
            jax.ShapeDtypeStruct((n, size), x.dtype),
        ],
    )(x, sel, mask, evg2, mu2, sig2)
    return out, probs
